# baseline (speedup 1.0000x reference)
import functools

import jax
import jax.numpy as jnp
from jax import lax
from jax.experimental import pallas as pl
from jax.experimental.pallas import tpu as pltpu
from jax.experimental.pallas import tpu_sc as plsc

B = 4
S = 8192
D = 1024

NC = 2
NS = 16
NW = NC * NS
S_PER_W = S // NW
K = 32
NCHUNK = S_PER_W // K
CW = K * D
LANES = 16
UNROLL = 8


def _body(x_hbm, pe_hbm, out_hbm, pe_buf, x_buf):
    wid = lax.axis_index("s") * NC + lax.axis_index("c")
    base = wid * (S_PER_W * D)
    for chunk in range(NCHUNK):
        off = base + chunk * CW
        pltpu.sync_copy(pe_hbm.at[pl.ds(off, CW)], pe_buf)
        for b in range(B):
            xoff = b * (S * D) + off
            pltpu.sync_copy(x_hbm.at[pl.ds(xoff, CW)], x_buf)

            def add_body(i, carry):
                o = i * (LANES * UNROLL)
                for u in range(UNROLL):
                    sl = pl.ds(o + u * LANES, LANES)
                    x_buf[sl] = x_buf[sl] + pe_buf[sl]
                return carry

            lax.fori_loop(0, CW // (LANES * UNROLL), add_body, 0)
            pltpu.sync_copy(x_buf, out_hbm.at[pl.ds(xoff, CW)])


_mesh = plsc.VectorSubcoreMesh(core_axis_name="c", subcore_axis_name="s")

_sc_add = pl.kernel(
    _body,
    mesh=_mesh,
    out_type=jax.ShapeDtypeStruct((B * S * D,), jnp.float32),
    scratch_types=[
        pltpu.VMEM((CW,), jnp.float32),
        pltpu.VMEM((CW,), jnp.float32),
    ],
)


def kernel(x, pos_embedding):
    out_flat = _sc_add(x.reshape(-1), pos_embedding.reshape(-1))
    return out_flat.reshape(x.shape)


# speedup vs baseline: 1.2171x; 1.2171x over previous
import functools

import jax
import jax.numpy as jnp
from jax import lax
from jax.experimental import pallas as pl
from jax.experimental.pallas import tpu as pltpu
from jax.experimental.pallas import tpu_sc as plsc

B = 4
S = 8192
D = 1024

NC = 2
NS = 16
NW = NC * NS
S_PER_W = S // NW
K = 16
NCHUNK = S_PER_W // K
CW = K * D
NSTEP = NCHUNK * B
LANES = 16
UNROLL = 8


def _body(x_hbm, pe_hbm, out_hbm,
          pe0, pe1, xin0, xin1, xout0, xout1,
          pe_sem0, pe_sem1, ld_sem0, ld_sem1, st_sem0, st_sem1):
    wid = lax.axis_index("s") * NC + lax.axis_index("c")
    base = wid * (S_PER_W * D)

    pe_bufs = [pe0, pe1]
    xins = [xin0, xin1]
    xouts = [xout0, xout1]
    pe_sems = [pe_sem0, pe_sem1]
    ld_sems = [ld_sem0, ld_sem1]
    st_sems = [st_sem0, st_sem1]

    def pe_load(c):
        return pltpu.async_copy(
            pe_hbm.at[pl.ds(base + c * CW, CW)], pe_bufs[c % 2], pe_sems[c % 2])

    def x_load(t):
        c, b = divmod(t, B)
        off = b * (S * D) + base + c * CW
        return pltpu.async_copy(
            x_hbm.at[pl.ds(off, CW)], xins[t % 2], ld_sems[t % 2])

    def x_store(t):
        c, b = divmod(t, B)
        off = b * (S * D) + base + c * CW
        return pltpu.async_copy(
            xouts[t % 2], out_hbm.at[pl.ds(off, CW)], st_sems[t % 2])

    pe_h = [pe_load(0), None]
    ld_h = [x_load(0), None]
    st_h = [None, None]

    for t in range(NSTEP):
        c, b = divmod(t, B)
        p = t % 2
        if t + 1 < NSTEP:
            ld_h[(t + 1) % 2] = x_load(t + 1)
        if b == 0 and c + 1 < NCHUNK:
            pe_h[(c + 1) % 2] = pe_load(c + 1)
        ld_h[p].wait()
        if b == 0:
            pe_h[c % 2].wait()
        if st_h[p] is not None:
            st_h[p].wait()

        xin, xout, pe_buf = xins[p], xouts[p], pe_bufs[c % 2]

        def add_body(i, carry):
            o = i * (LANES * UNROLL)
            for u in range(UNROLL):
                sl = pl.ds(o + u * LANES, LANES)
                xout[sl] = xin[sl] + pe_buf[sl]
            return carry

        lax.fori_loop(0, CW // (LANES * UNROLL), add_body, 0)
        st_h[p] = x_store(t)

    st_h[(NSTEP - 2) % 2].wait()
    st_h[(NSTEP - 1) % 2].wait()


_mesh = plsc.VectorSubcoreMesh(core_axis_name="c", subcore_axis_name="s")

_sc_add = pl.kernel(
    _body,
    mesh=_mesh,
    out_type=jax.ShapeDtypeStruct((B * S * D,), jnp.float32),
    scratch_types=[
        pltpu.VMEM((CW,), jnp.float32),
        pltpu.VMEM((CW,), jnp.float32),
        pltpu.VMEM((CW,), jnp.float32),
        pltpu.VMEM((CW,), jnp.float32),
        pltpu.VMEM((CW,), jnp.float32),
        pltpu.VMEM((CW,), jnp.float32),
        pltpu.SemaphoreType.DMA,
        pltpu.SemaphoreType.DMA,
        pltpu.SemaphoreType.DMA,
        pltpu.SemaphoreType.DMA,
        pltpu.SemaphoreType.DMA,
        pltpu.SemaphoreType.DMA,
    ],
)


def kernel(x, pos_embedding):
    out_flat = _sc_add(x.reshape(-1), pos_embedding.reshape(-1))
    return out_flat.reshape(x.shape)


# speedup vs baseline: 2.6381x; 2.1676x over previous
import functools

import jax
import jax.numpy as jnp
from jax import lax
from jax.experimental import pallas as pl
from jax.experimental.pallas import tpu as pltpu
from jax.experimental.pallas import tpu_sc as plsc

B = 4
S = 8192
D = 1024

NC = 2
NS = 16
NW = NC * NS
S_PER_W = S // NW
K = 16
NCHUNK = S_PER_W // K
NSTEP = NCHUNK * B
LANES = 16


def _body(x_hbm, pe_hbm, out_hbm,
          pe0, pe1, xin0, xin1, xout0, xout1,
          pe_sem0, pe_sem1, ld_sem0, ld_sem1, st_sem0, st_sem1):
    wid = lax.axis_index("s") * NC + lax.axis_index("c")
    base = wid * S_PER_W

    pe_bufs = [pe0, pe1]
    xins = [xin0, xin1]
    xouts = [xout0, xout1]
    pe_sems = [pe_sem0, pe_sem1]
    ld_sems = [ld_sem0, ld_sem1]
    st_sems = [st_sem0, st_sem1]

    def pe_load(c):
        return pltpu.async_copy(
            pe_hbm.at[pl.ds(base + c * K, K)], pe_bufs[c % 2], pe_sems[c % 2])

    def x_load(t):
        c, b = divmod(t, B)
        row = b * S + base + c * K
        return pltpu.async_copy(
            x_hbm.at[pl.ds(row, K)], xins[t % 2], ld_sems[t % 2])

    def x_store(t):
        c, b = divmod(t, B)
        row = b * S + base + c * K
        return pltpu.async_copy(
            xouts[t % 2], out_hbm.at[pl.ds(row, K)], st_sems[t % 2])

    pe_h = [pe_load(0), None]
    ld_h = [x_load(0), None]
    st_h = [None, None]

    for t in range(NSTEP):
        c, b = divmod(t, B)
        p = t % 2
        if t + 1 < NSTEP:
            ld_h[(t + 1) % 2] = x_load(t + 1)
        if b == 0 and c + 1 < NCHUNK:
            pe_h[(c + 1) % 2] = pe_load(c + 1)
        ld_h[p].wait()
        if b == 0:
            pe_h[c % 2].wait()
        if st_h[p] is not None:
            st_h[p].wait()

        xin, xout, pe_buf = xins[p], xouts[p], pe_bufs[c % 2]

        def add_body(i, carry):
            sl = pl.ds(i * LANES, LANES)
            for r in range(K):
                xout[r, sl] = xin[r, sl] + pe_buf[r, sl]
            return carry

        lax.fori_loop(0, D // LANES, add_body, 0)
        st_h[p] = x_store(t)

    st_h[(NSTEP - 2) % 2].wait()
    st_h[(NSTEP - 1) % 2].wait()


_mesh = plsc.VectorSubcoreMesh(core_axis_name="c", subcore_axis_name="s")

_sc_add = pl.kernel(
    _body,
    mesh=_mesh,
    out_type=jax.ShapeDtypeStruct((B * S, D), jnp.float32),
    scratch_types=[
        pltpu.VMEM((K, D), jnp.float32),
        pltpu.VMEM((K, D), jnp.float32),
        pltpu.VMEM((K, D), jnp.float32),
        pltpu.VMEM((K, D), jnp.float32),
        pltpu.VMEM((K, D), jnp.float32),
        pltpu.VMEM((K, D), jnp.float32),
        pltpu.SemaphoreType.DMA,
        pltpu.SemaphoreType.DMA,
        pltpu.SemaphoreType.DMA,
        pltpu.SemaphoreType.DMA,
        pltpu.SemaphoreType.DMA,
        pltpu.SemaphoreType.DMA,
    ],
    compiler_params=pltpu.CompilerParams(use_tc_tiling_on_sc=True),
)


def kernel(x, pos_embedding):
    out2d = _sc_add(x.reshape(B * S, D), pos_embedding)
    return out2d.reshape(x.shape)


# speedup vs baseline: 4.4613x; 1.6911x over previous
import functools

import jax
import jax.numpy as jnp
from jax import lax
from jax.experimental import pallas as pl
from jax.experimental.pallas import tpu as pltpu
from jax.experimental.pallas import tpu_sc as plsc

B = 4
S = 8192
D = 1024

NC = 2
NS = 16
NW = NC * NS
S_PER_W = S // NW
K = 16
NCHUNK = S_PER_W // K
NSTEP = NCHUNK * B
LANES = 16


def _body(x_hbm, pe_hbm, out_hbm,
          pe0, pe1, xin0, xin1, xout0, xout1,
          pe_sem0, pe_sem1, ld_sem0, ld_sem1, st_sem0, st_sem1):
    wid = lax.axis_index("s") * NC + lax.axis_index("c")
    base = wid * S_PER_W

    pe_bufs = [pe0, pe1]
    xins = [xin0, xin1]
    xouts = [xout0, xout1]
    pe_sems = [pe_sem0, pe_sem1]
    ld_sems = [ld_sem0, ld_sem1]
    st_sems = [st_sem0, st_sem1]

    def pe_load(c):
        return pltpu.async_copy(
            pe_hbm.at[pl.ds(base + c * K, K)], pe_bufs[c % 2], pe_sems[c % 2])

    def x_load(t):
        c, b = divmod(t, B)
        row = b * S + base + c * K
        return pltpu.async_copy(
            x_hbm.at[pl.ds(row, K)], xins[t % 2], ld_sems[t % 2])

    def x_store(t):
        c, b = divmod(t, B)
        row = b * S + base + c * K
        return pltpu.async_copy(
            xouts[t % 2], out_hbm.at[pl.ds(row, K)], st_sems[t % 2])

    pe_h = [pe_load(0), None]
    ld_h = [x_load(0), None]
    st_h = [None, None]

    for t in range(NSTEP):
        c, b = divmod(t, B)
        p = t % 2
        if t + 1 < NSTEP:
            ld_h[(t + 1) % 2] = x_load(t + 1)
        if b == 0 and c + 1 < NCHUNK:
            pe_h[(c + 1) % 2] = pe_load(c + 1)
        ld_h[p].wait()
        if b == 0:
            pe_h[c % 2].wait()
        if st_h[p] is not None:
            st_h[p].wait()

        xin, xout, pe_buf = xins[p], xouts[p], pe_bufs[c % 2]

        def add_body(i, carry):
            sl = pl.ds(i * LANES, LANES)
            for r in range(K):
                xout[r, sl] = xin[r, sl] + pe_buf[r, sl]
            return carry

        lax.fori_loop(0, D // LANES, add_body, 0)
        st_h[p] = x_store(t)

    st_h[(NSTEP - 2) % 2].wait()
    st_h[(NSTEP - 1) % 2].wait()


_mesh = plsc.VectorSubcoreMesh(core_axis_name="c", subcore_axis_name="s")

_sc_add = pl.kernel(
    _body,
    mesh=_mesh,
    out_type=jax.ShapeDtypeStruct((B * S, D), jnp.float32),
    scratch_types=[
        pltpu.VMEM((K, D), jnp.float32),
        pltpu.VMEM((K, D), jnp.float32),
        pltpu.VMEM((K, D), jnp.float32),
        pltpu.VMEM((K, D), jnp.float32),
        pltpu.VMEM((K, D), jnp.float32),
        pltpu.VMEM((K, D), jnp.float32),
        pltpu.SemaphoreType.DMA,
        pltpu.SemaphoreType.DMA,
        pltpu.SemaphoreType.DMA,
        pltpu.SemaphoreType.DMA,
        pltpu.SemaphoreType.DMA,
        pltpu.SemaphoreType.DMA,
    ],
    compiler_params=pltpu.CompilerParams(use_tc_tiling_on_sc=True),
)


BS_TC = 512


def _tc_body(x_ref, pe_ref, out_ref):
    out_ref[0] = x_ref[0] + pe_ref[...]


_tc_add = pl.pallas_call(
    _tc_body,
    grid=(S // BS_TC, B),
    in_specs=[
        pl.BlockSpec((1, BS_TC, D), lambda s, b: (b, s, 0)),
        pl.BlockSpec((BS_TC, D), lambda s, b: (s, 0)),
    ],
    out_specs=pl.BlockSpec((1, BS_TC, D), lambda s, b: (b, s, 0)),
    out_shape=jax.ShapeDtypeStruct((B, S, D), jnp.float32),
)


def kernel(x, pos_embedding):
    return _tc_add(x, pos_embedding)
